# baseline (speedup 1.0000x reference)
import functools

import jax
import jax.numpy as jnp
from jax import lax
from jax.experimental import pallas as pl
from jax.experimental.pallas import tpu as pltpu
from jax.experimental.pallas import tpu_sc as plsc

_N = 4194304
_COLS = 1024
_ROWS = _N // _COLS

_SC_CHUNK = 16384
_NW = 32
_SC_K = 4
_SCN = _NW * _SC_CHUNK * _SC_K
_SC_PW = _SCN // _NW
_SC_ROWS = _SCN // _COLS

_TC_CHROWS = 512
_TC_ROWS = _ROWS - _SC_ROWS
_TC_NCH = _TC_ROWS // _TC_CHROWS
_TC_DEPTH = 4
_L = 16

_mesh = plsc.VectorSubcoreMesh(core_axis_name="c", subcore_axis_name="s")


@functools.partial(
    pl.kernel,
    out_type=jax.ShapeDtypeStruct((_NW, 3, _L), jnp.float32),
    mesh=_mesh,
    scratch_types=[
        pltpu.VMEM((2, _SC_CHUNK), jnp.float32),
        pltpu.VMEM((2, _SC_CHUNK), jnp.int32),
        pltpu.VMEM((3, _L), jnp.float32),
        pltpu.SemaphoreType.DMA,
        pltpu.SemaphoreType.DMA,
        pltpu.SemaphoreType.DMA,
        pltpu.SemaphoreType.DMA,
    ],
)
def _sc_partial(y_hbm, s_hbm, out_hbm, ybuf, sbuf, accv,
                sem_y0, sem_y1, sem_s0, sem_s1):
    wid = lax.axis_index("s") * 2 + lax.axis_index("c")
    base = wid * _SC_PW
    sems_y = (sem_y0, sem_y1)
    sems_s = (sem_s0, sem_s1)

    def start(k, slot):
        off = base + k * _SC_CHUNK
        cy = pltpu.make_async_copy(
            y_hbm.at[pl.ds(off, _SC_CHUNK)], ybuf.at[slot], sems_y[slot])
        cs = pltpu.make_async_copy(
            s_hbm.at[pl.ds(off, _SC_CHUNK)], sbuf.at[slot], sems_s[slot])
        cy.start()
        cs.start()
        return cy, cs

    def chunk_reduce(acc, slot):
        yb = ybuf.at[slot]
        sb = sbuf.at[slot]

        def body(i, acc):
            ay, ays, asf = acc
            yv = yb[pl.ds(i * _L, _L)]
            sv = sb[pl.ds(i * _L, _L)]
            ay = ay + yv
            ays = ays + jnp.where(sv == 1, yv, jnp.float32(0.0))
            asf = asf + sv.astype(jnp.float32)
            return (ay, ays, asf)

        return lax.fori_loop(0, _SC_CHUNK // _L, body, acc, unroll=8)

    zero = jnp.zeros((_L,), jnp.float32)
    acc = (zero, zero, zero)
    pending = start(0, 0)
    for k in range(_SC_K):
        slot = k % 2
        cy, cs = pending
        cy.wait()
        cs.wait()
        if k + 1 < _SC_K:
            pending = start(k + 1, (k + 1) % 2)
        acc = chunk_reduce(acc, slot)

    accv[0] = acc[0]
    accv[1] = acc[1]
    accv[2] = acc[2]
    pltpu.sync_copy(accv, out_hbm.at[wid])


def _tc_body(y_hbm, s_hbm, oy_ref, oys_ref, os_ref,
             ybuf, sbuf, sems_y, sems_s):
    def start(k):
        slot = k % _TC_DEPTH
        row0 = _SC_ROWS + k * _TC_CHROWS
        cy = pltpu.make_async_copy(
            y_hbm.at[pl.ds(row0, _TC_CHROWS), :], ybuf.at[slot],
            sems_y.at[slot])
        cs = pltpu.make_async_copy(
            s_hbm.at[pl.ds(row0, _TC_CHROWS), :], sbuf.at[slot],
            sems_s.at[slot])
        cy.start(priority=0)
        cs.start(priority=1)
        return cy, cs

    pending = [start(k) for k in range(min(_TC_DEPTH, _TC_NCH))]

    zero = jnp.zeros((8, _COLS), jnp.float32)
    acc = (zero, zero, zero)
    for k in range(_TC_NCH):
        slot = k % _TC_DEPTH
        cy, cs = pending[slot]
        cy.wait()
        cs.wait()

        def body(i, a, _slot=slot):
            ay, ays, asf = a
            yv = ybuf[_slot, pl.ds(i * 8, 8), :]
            sv = sbuf[_slot, pl.ds(i * 8, 8), :]
            ay = ay + yv
            ays = ays + jnp.where(sv == 1, yv, jnp.float32(0.0))
            asf = asf + sv.astype(jnp.float32)
            return (ay, ays, asf)

        acc = lax.fori_loop(0, _TC_CHROWS // 8, body, acc, unroll=2)
        if k + _TC_DEPTH < _TC_NCH:
            pending[slot] = start(k + _TC_DEPTH)

    oy_ref[0, 0] = jnp.sum(acc[0])
    oys_ref[0, 0] = jnp.sum(acc[1])
    os_ref[0, 0] = jnp.sum(acc[2])


_tc_reduce = pl.pallas_call(
    _tc_body,
    in_specs=[
        pl.BlockSpec(memory_space=pl.ANY),
        pl.BlockSpec(memory_space=pl.ANY),
    ],
    out_specs=[
        pl.BlockSpec(memory_space=pltpu.SMEM),
        pl.BlockSpec(memory_space=pltpu.SMEM),
        pl.BlockSpec(memory_space=pltpu.SMEM),
    ],
    out_shape=[
        jax.ShapeDtypeStruct((1, 1), jnp.float32),
        jax.ShapeDtypeStruct((1, 1), jnp.float32),
        jax.ShapeDtypeStruct((1, 1), jnp.float32),
    ],
    scratch_shapes=[
        pltpu.VMEM((_TC_DEPTH, _TC_CHROWS, _COLS), jnp.float32),
        pltpu.VMEM((_TC_DEPTH, _TC_CHROWS, _COLS), jnp.int32),
        pltpu.SemaphoreType.DMA((_TC_DEPTH,)),
        pltpu.SemaphoreType.DMA((_TC_DEPTH,)),
    ],
)


def kernel(y_pred, s):
    yf = y_pred.reshape(-1)
    sf = s.reshape(-1)
    sc_parts = _sc_partial(yf, sf)
    y2 = yf.reshape(_ROWS, _COLS)
    s2 = sf.reshape(_ROWS, _COLS)
    tcy, tcys, tccnt = _tc_reduce(y2, s2)
    sc_sums = jnp.sum(sc_parts, axis=(0, 2))
    sum_y = sc_sums[0] + tcy[0, 0]
    sum_ys = sc_sums[1] + tcys[0, 0]
    c1 = sc_sums[2] + tccnt[0, 0]
    c0 = jnp.float32(_N) - c1
    mean1 = sum_ys / c1
    mean0 = (sum_y - sum_ys) / c0
    return jnp.abs(mean0 - mean1)


# speedup vs baseline: 1.2811x; 1.2811x over previous
import functools

import jax
import jax.numpy as jnp
from jax.experimental import pallas as pl
from jax.experimental.pallas import tpu as pltpu

_N = 4194304
_COLS = 1024
_ROWS = _N // _COLS
_CHROWS = 128
_NCH = _ROWS // _CHROWS
_DEPTH = 16


def _tc_body(y_hbm, s_hbm, oy_ref, oys_ref, os_ref,
             ybuf, sbuf, sems_y, sems_s, accy, accys, accs):
    def start(k):
        slot = k % _DEPTH
        cy = pltpu.make_async_copy(
            y_hbm.at[pl.ds(k * _CHROWS, _CHROWS), :], ybuf.at[slot],
            sems_y.at[slot])
        cs = pltpu.make_async_copy(
            s_hbm.at[pl.ds(k * _CHROWS, _CHROWS), :], sbuf.at[slot],
            sems_s.at[slot])
        cy.start(priority=0)
        cs.start(priority=1)
        return cy, cs

    pending = [start(k) for k in range(_DEPTH)]
    accy[...] = jnp.zeros((8, _COLS), jnp.float32)
    accys[...] = jnp.zeros((8, _COLS), jnp.float32)
    accs[...] = jnp.zeros((8, _COLS), jnp.float32)

    for k in range(_NCH):
        slot = k % _DEPTH
        cy, cs = pending[slot]
        cy.wait()
        cs.wait()
        yv = ybuf[slot]
        sv = sbuf[slot]
        ysel = jnp.where(sv == 1, yv, jnp.float32(0.0))
        sf = sv.astype(jnp.float32)
        accy[...] += jnp.sum(yv.reshape(-1, 8, _COLS), axis=0)
        accys[...] += jnp.sum(ysel.reshape(-1, 8, _COLS), axis=0)
        accs[...] += jnp.sum(sf.reshape(-1, 8, _COLS), axis=0)
        if k + _DEPTH < _NCH:
            pending[slot] = start(k + _DEPTH)

    oy_ref[0, 0] = jnp.sum(accy[...])
    oys_ref[0, 0] = jnp.sum(accys[...])
    os_ref[0, 0] = jnp.sum(accs[...])


_tc_reduce = pl.pallas_call(
    _tc_body,
    in_specs=[
        pl.BlockSpec(memory_space=pl.ANY),
        pl.BlockSpec(memory_space=pl.ANY),
    ],
    out_specs=[
        pl.BlockSpec(memory_space=pltpu.SMEM),
        pl.BlockSpec(memory_space=pltpu.SMEM),
        pl.BlockSpec(memory_space=pltpu.SMEM),
    ],
    out_shape=[
        jax.ShapeDtypeStruct((1, 1), jnp.float32),
        jax.ShapeDtypeStruct((1, 1), jnp.float32),
        jax.ShapeDtypeStruct((1, 1), jnp.float32),
    ],
    scratch_shapes=[
        pltpu.VMEM((_DEPTH, _CHROWS, _COLS), jnp.float32),
        pltpu.VMEM((_DEPTH, _CHROWS, _COLS), jnp.int32),
        pltpu.SemaphoreType.DMA((_DEPTH,)),
        pltpu.SemaphoreType.DMA((_DEPTH,)),
        pltpu.VMEM((8, _COLS), jnp.float32),
        pltpu.VMEM((8, _COLS), jnp.float32),
        pltpu.VMEM((8, _COLS), jnp.float32),
    ],
)


def kernel(y_pred, s):
    y2 = y_pred.reshape(_ROWS, _COLS)
    s2 = s.reshape(_ROWS, _COLS)
    sy, sys_, cnt1 = _tc_reduce(y2, s2)
    sum_y = sy[0, 0]
    sum_ys = sys_[0, 0]
    c1 = cnt1[0, 0]
    c0 = jnp.float32(_N) - c1
    mean1 = sum_ys / c1
    mean0 = (sum_y - sum_ys) / c0
    return jnp.abs(mean0 - mean1)
